# Initial kernel scaffold; baseline (speedup 1.0000x reference)
import jax
import jax.numpy as jnp
from jax.experimental import pallas as pl


def kernel(x, pe, edge_index, edge_attr, batch, params):
    raise NotImplementedError("write your pallas kernel here")


# speedup vs baseline: 3.4668x; 3.4668x over previous
import functools

import jax
import jax.numpy as jnp
from jax import lax
from jax.experimental import pallas as pl
from jax.experimental.pallas import tpu as pltpu
from jax.experimental.pallas import tpu_sc as plsc

N = 50000
E = 800000
G = 2048
HID = 128
PE_D = 16
EPS = 1e-5
NB = 1000
NBLK = N // NB
EB = 128
NTILE = 16
EPT = 50048
E_PAD = NTILE * EPT
EBLK = 782
EBS = 1024

R_LO = (0, 12504, 25008, 37512)
R_SZ = (12504, 12504, 12504, 12488)
TRASH = 12504
AGG_ROWS = 12512
SRS = 784
SRS_OFF_L = 15 * SRS

f32 = jnp.float32
i32 = jnp.int32
_PH = lax.Precision.HIGHEST


def _dot(a, b):
    return jnp.dot(a, b, precision=_PH, preferred_element_type=f32)


def _dg(a, b, dn):
    return lax.dot_general(a, b, dn, precision=_PH, preferred_element_type=f32)


def _dotb(a, b):
    return jnp.dot(a.astype(jnp.bfloat16), b.astype(jnp.bfloat16),
                   preferred_element_type=f32)


def _eidx_body(s_ref, a_ref, d_ref, o_ref, d0_ref, d1_ref, d2_ref, d3_ref):
    o_ref[...] = a_ref[...] * N + s_ref[...]
    d = d_ref[...]
    outs = (d0_ref, d1_ref, d2_ref, d3_ref)
    for r in range(4):
        inr = (d >= R_LO[r]) & (d < R_LO[r] + R_SZ[r])
        outs[r][...] = jnp.where(inr, d - R_LO[r], TRASH)


_eidx_call = pl.pallas_call(
    _eidx_body,
    grid=(EBLK,),
    in_specs=[pl.BlockSpec((1, 1, EBS), lambda i: (i, 0, 0))] * 3,
    out_specs=[pl.BlockSpec((1, 1, EBS), lambda i: (i, 0, 0))] * 5,
    out_shape=[jax.ShapeDtypeStruct((EBLK, 1, EBS), i32)] * 5,
)


def _pestats_body(pe_ref, W1_ref, b1_ref, mu_ref, var_ref, s_ref, C_ref):
    i = pl.program_id(0)

    @pl.when(i == 0)
    def _():
        s_ref[...] = jnp.zeros_like(s_ref)
        C_ref[...] = jnp.zeros_like(C_ref)

    blk = pe_ref[...]
    s_ref[...] += jnp.sum(blk, axis=0, keepdims=True)
    C_ref[...] += _dg(blk, blk, (((0,), (0,)), ((), ())))

    @pl.when(i == NBLK - 1)
    def _():
        m = s_ref[...] / N
        W1 = W1_ref[...]
        mu_ref[...] = _dot(m, W1) + b1_ref[...]
        cov = C_ref[...] / N - _dg(m, m, (((0,), (0,)), ((), ())))
        t = _dot(cov, W1)
        var_ref[...] = jnp.sum(W1 * t, axis=0, keepdims=True)


_pestats_call = pl.pallas_call(
    _pestats_body,
    grid=(NBLK,),
    in_specs=[
        pl.BlockSpec((NB, PE_D), lambda i: (i, 0)),
        pl.BlockSpec((PE_D, HID), lambda i: (0, 0)),
        pl.BlockSpec((1, HID), lambda i: (0, 0)),
    ],
    out_specs=[
        pl.BlockSpec((1, HID), lambda i: (0, 0)),
        pl.BlockSpec((1, HID), lambda i: (0, 0)),
    ],
    out_shape=[jax.ShapeDtypeStruct((1, HID), f32),
               jax.ShapeDtypeStruct((1, HID), f32)],
    scratch_shapes=[pltpu.VMEM((1, PE_D), f32), pltpu.VMEM((PE_D, PE_D), f32)],
)


def _prep_body(x_ref, pe_ref, A_ref, W1_ref, b1_ref, sc_ref, sh_ref,
               W2_ref, b2_ref, Wb_ref, inb_ref, et_ref, h0_ref, tab_ref):
    a = pl.program_id(0)
    xb = x_ref[...].reshape(1, NB)
    oh = (lax.broadcasted_iota(i32, (28, NB), 0) == xb).astype(f32)
    emb = _dg(oh, A_ref[...], (((0,), (0,)), ((), ())))
    t = _dotb(pe_ref[...], W1_ref[...]) + b1_ref[...]
    t = jnp.maximum(t * sc_ref[...] + sh_ref[...], 0.0)
    pe_out = _dotb(t, W2_ref[...]) + b2_ref[...]
    h0 = emb + _dotb(pe_out, Wb_ref[...]) + inb_ref[...]
    h0_ref[...] = h0
    ea = et_ref[pl.ds(a, 1), :]
    tab_ref[...] = jnp.maximum(h0 + ea, 0.0)


_prep_call = pl.pallas_call(
    _prep_body,
    grid=(4, NBLK),
    in_specs=[
        pl.BlockSpec((1, 1, NB), lambda a, i: (i, 0, 0)),
        pl.BlockSpec((NB, PE_D), lambda a, i: (i, 0)),
        pl.BlockSpec((28, HID), lambda a, i: (0, 0)),
        pl.BlockSpec((PE_D, HID), lambda a, i: (0, 0)),
        pl.BlockSpec((1, HID), lambda a, i: (0, 0)),
        pl.BlockSpec((1, HID), lambda a, i: (0, 0)),
        pl.BlockSpec((1, HID), lambda a, i: (0, 0)),
        pl.BlockSpec((HID, HID), lambda a, i: (0, 0)),
        pl.BlockSpec((1, HID), lambda a, i: (0, 0)),
        pl.BlockSpec((HID, HID), lambda a, i: (0, 0)),
        pl.BlockSpec((1, HID), lambda a, i: (0, 0)),
        pl.BlockSpec((4, HID), lambda a, i: (0, 0)),
    ],
    out_specs=[
        pl.BlockSpec((NB, HID), lambda a, i: (i, 0)),
        pl.BlockSpec((NB, HID), lambda a, i: (a * NBLK + i, 0)),
    ],
    out_shape=[jax.ShapeDtypeStruct((N, HID), f32),
               jax.ShapeDtypeStruct((4 * N, HID), f32)],
)


def _sc_body(tab, idx_hbm, d0, d1, d2, d3, bounds_hbm, zeros_hbm,
             out_hbm, idx_v, dst_v, rows_v, bnd_v, agg, gsem):
    cid = lax.axis_index("c")
    sid = lax.axis_index("s")

    pltpu.sync_copy(bounds_hbm, bnd_v)

    def sel(k):
        return bnd_v[...][k]

    def stripe_copy(src, dst, src_off, dst_off, last_len):
        @pl.when(sid < NTILE - 1)
        def _():
            b = pl.multiple_of(sid * SRS, 8)
            pltpu.sync_copy(src.at[pl.ds(src_off + b, SRS)],
                            dst.at[pl.ds(dst_off + b, SRS)])

        @pl.when(sid == NTILE - 1)
        def _():
            pltpu.sync_copy(src.at[pl.ds(src_off + SRS_OFF_L, last_len)],
                            dst.at[pl.ds(dst_off + SRS_OFF_L, last_len)])

    def run_range(dr_hbm, r):
        lo = sel(2 * r)
        hi = sel(2 * r + 1)
        lo_a = (lo // 8) * 8
        nb_tot = (hi - lo_a + EB - 1) // EB
        nb = (nb_tot + NTILE - 1) // NTILE
        start = lo_a + sid * (nb * EB)

        last = R_SZ[r] - SRS_OFF_L
        stripe_copy(zeros_hbm, agg, 0, 0, last)
        plsc.subcore_barrier()

        def body(j, carry):
            base = pl.multiple_of(
                jnp.minimum(start + j * EB, E_PAD - EB), 8)
            pltpu.sync_copy(idx_hbm.at[pl.ds(base, EB)], idx_v)
            pltpu.sync_copy(dr_hbm.at[pl.ds(base, EB)], dst_v)
            pltpu.async_copy(tab.at[idx_v], rows_v, gsem).wait()
            pltpu.sync_copy(rows_v, agg.at[dst_v], add=True)
            return carry

        lax.fori_loop(0, nb, body, 0)
        plsc.subcore_barrier()
        stripe_copy(agg, out_hbm, 0, R_LO[r], last)
        plsc.subcore_barrier()

    @pl.when(cid == 0)
    def _():
        run_range(d0, 0)
        run_range(d1, 1)

    @pl.when(cid == 1)
    def _():
        run_range(d2, 2)
        run_range(d3, 3)


_sc_call = functools.partial(
    pl.kernel,
    out_type=jax.ShapeDtypeStruct((N, HID), f32),
    mesh=plsc.VectorSubcoreMesh(core_axis_name="c", subcore_axis_name="s"),
    scratch_types=[
        pltpu.VMEM((EB,), i32),
        pltpu.VMEM((EB,), i32),
        pltpu.VMEM((EB, HID), f32),
        pltpu.VMEM((16,), i32),
        pltpu.VMEM_SHARED((AGG_ROWS, HID), f32),
        pltpu.SemaphoreType.DMA,
    ],
)(_sc_body)


def _dense_body(h_ref, agg_ref, W1_ref, b1_ref, W2_ref, b2_ref,
                z_ref, s1_ref, s2_ref):
    i = pl.program_id(0)

    @pl.when(i == 0)
    def _():
        s1_ref[...] = jnp.zeros_like(s1_ref)
        s2_ref[...] = jnp.zeros_like(s2_ref)

    inp = h_ref[...] + agg_ref[...]
    t = jnp.maximum(_dotb(inp, W1_ref[...]) + b1_ref[...], 0.0)
    z = _dotb(t, W2_ref[...]) + b2_ref[...]
    z_ref[...] = z
    s1_ref[...] += jnp.sum(z, axis=0, keepdims=True)
    s2_ref[...] += jnp.sum(z * z, axis=0, keepdims=True)


_dense_call = pl.pallas_call(
    _dense_body,
    grid=(NBLK,),
    in_specs=[
        pl.BlockSpec((NB, HID), lambda i: (i, 0)),
        pl.BlockSpec((NB, HID), lambda i: (i, 0)),
        pl.BlockSpec((HID, HID), lambda i: (0, 0)),
        pl.BlockSpec((1, HID), lambda i: (0, 0)),
        pl.BlockSpec((HID, HID), lambda i: (0, 0)),
        pl.BlockSpec((1, HID), lambda i: (0, 0)),
    ],
    out_specs=[
        pl.BlockSpec((NB, HID), lambda i: (i, 0)),
        pl.BlockSpec((1, HID), lambda i: (0, 0)),
        pl.BlockSpec((1, HID), lambda i: (0, 0)),
    ],
    out_shape=[jax.ShapeDtypeStruct((N, HID), f32),
               jax.ShapeDtypeStruct((1, HID), f32),
               jax.ShapeDtypeStruct((1, HID), f32)],
)


def _bn_tab_body(z_ref, s1_ref, s2_ref, g_ref, b_ref, et_ref,
                 h_ref, tab_ref):
    a = pl.program_id(0)
    mu = s1_ref[...] / N
    var = s2_ref[...] / N - mu * mu
    inv = (1.0 / jnp.sqrt(var + EPS)) * g_ref[...]
    hn = jnp.maximum((z_ref[...] - mu) * inv + b_ref[...], 0.0)
    h_ref[...] = hn
    ea = et_ref[pl.ds(a, 1), :]
    tab_ref[...] = jnp.maximum(hn + ea, 0.0)


_bn_tab_call = pl.pallas_call(
    _bn_tab_body,
    grid=(4, NBLK),
    in_specs=[
        pl.BlockSpec((NB, HID), lambda a, i: (i, 0)),
        pl.BlockSpec((1, HID), lambda a, i: (0, 0)),
        pl.BlockSpec((1, HID), lambda a, i: (0, 0)),
        pl.BlockSpec((1, HID), lambda a, i: (0, 0)),
        pl.BlockSpec((1, HID), lambda a, i: (0, 0)),
        pl.BlockSpec((4, HID), lambda a, i: (0, 0)),
    ],
    out_specs=[
        pl.BlockSpec((NB, HID), lambda a, i: (i, 0)),
        pl.BlockSpec((NB, HID), lambda a, i: (a * NBLK + i, 0)),
    ],
    out_shape=[jax.ShapeDtypeStruct((N, HID), f32),
               jax.ShapeDtypeStruct((4 * N, HID), f32)],
)


def _bn_final_body(z_ref, s1_ref, s2_ref, g_ref, b_ref, h_ref):
    mu = s1_ref[...] / N
    var = s2_ref[...] / N - mu * mu
    inv = (1.0 / jnp.sqrt(var + EPS)) * g_ref[...]
    h_ref[...] = jnp.maximum((z_ref[...] - mu) * inv + b_ref[...], 0.0)


_bn_final_call = pl.pallas_call(
    _bn_final_body,
    grid=(NBLK,),
    in_specs=[
        pl.BlockSpec((NB, HID), lambda i: (i, 0)),
        pl.BlockSpec((1, HID), lambda i: (0, 0)),
        pl.BlockSpec((1, HID), lambda i: (0, 0)),
        pl.BlockSpec((1, HID), lambda i: (0, 0)),
        pl.BlockSpec((1, HID), lambda i: (0, 0)),
    ],
    out_specs=pl.BlockSpec((NB, HID), lambda i: (i, 0)),
    out_shape=jax.ShapeDtypeStruct((N, HID), f32),
)


def _readout_body(h_ref, b3_ref, W1_ref, b1_ref, W2_ref, b2_ref,
                  o_ref, xa_ref, cnt_ref):
    i = pl.program_id(0)

    @pl.when(i == 0)
    def _():
        xa_ref[...] = jnp.zeros_like(xa_ref)
        cnt_ref[...] = jnp.zeros_like(cnt_ref)

    ids = b3_ref[...].reshape(1, NB)
    oh = (lax.broadcasted_iota(i32, (G, NB), 0) == ids).astype(f32)
    xa_ref[...] += _dg(oh, h_ref[...], (((1,), (0,)), ((), ())))
    cnt_ref[...] += jnp.broadcast_to(
        jnp.sum(oh, axis=1, keepdims=True), (G, HID))

    @pl.when(i == NBLK - 1)
    def _():
        xa = xa_ref[...]
        cnt = jnp.maximum(cnt_ref[...], 1.0)
        g2 = jnp.concatenate([xa, xa / cnt], axis=1)
        t = jnp.maximum(_dotb(g2, W1_ref[...]) + b1_ref[...], 0.0)
        o_ref[...] = _dotb(t, W2_ref[...]) + b2_ref[...]


_readout_call = pl.pallas_call(
    _readout_body,
    grid=(NBLK,),
    in_specs=[
        pl.BlockSpec((NB, HID), lambda i: (i, 0)),
        pl.BlockSpec((1, 1, NB), lambda i: (i, 0, 0)),
        pl.BlockSpec((2 * HID, HID), lambda i: (0, 0)),
        pl.BlockSpec((1, HID), lambda i: (0, 0)),
        pl.BlockSpec((HID, HID), lambda i: (0, 0)),
        pl.BlockSpec((1, HID), lambda i: (0, 0)),
    ],
    out_specs=pl.BlockSpec((G, HID), lambda i: (0, 0)),
    out_shape=jax.ShapeDtypeStruct((G, HID), f32),
    scratch_shapes=[pltpu.VMEM((G, HID), f32), pltpu.VMEM((G, HID), f32)],
)


@jax.jit
def kernel(x, pe, edge_index, edge_attr, batch, params):
    inW = params['in_W']
    A = params['atom_emb'] @ inW[:64]
    etabs = [params['bond_emb'] @ c['We'] + c['be']
             for c in params['convs']]
    roW2 = jnp.zeros((HID, HID), f32).at[:, 0].set(params['ro_W2'][:, 0])
    rob2 = jnp.broadcast_to(params['ro_b2'], (HID,))[None, :]

    x3 = x[:, 0].reshape(NBLK, 1, NB).astype(i32)
    b3 = batch.reshape(NBLK, 1, NB).astype(i32)
    src = edge_index[0].astype(i32)
    dst = edge_index[1].astype(i32)
    att = edge_attr[:, 0].astype(i32)
    order = jnp.argsort(dst)
    src = src[order]
    dst = dst[order]
    att = att[order]
    pad = E_PAD - E
    src_p = jnp.concatenate([src, jnp.zeros((pad,), i32)])
    att_p = jnp.concatenate([att, jnp.zeros((pad,), i32)])
    dst_p = jnp.concatenate([dst, jnp.full((pad,), N, i32)])
    cuts = jnp.asarray([R_LO[1], R_LO[2], R_LO[3]], i32)
    b = jnp.searchsorted(dst_p, cuts).astype(i32)
    bounds = jnp.concatenate([
        jnp.zeros((1,), i32), b[0:1], b[0:1], b[1:2], b[1:2], b[2:3],
        b[2:3], jnp.full((1,), E_PAD, i32), jnp.zeros((8,), i32)])
    zeros_hbm = jnp.zeros((R_SZ[0], HID), f32)

    idx, d0, d1, d2, d3 = _eidx_call(src_p.reshape(EBLK, 1, EBS),
                                     att_p.reshape(EBLK, 1, EBS),
                                     dst_p.reshape(EBLK, 1, EBS))
    idx = idx.reshape(E_PAD)
    d0, d1, d2, d3 = (d.reshape(E_PAD) for d in (d0, d1, d2, d3))

    mu_h, var_h = _pestats_call(pe, params['pe_W1'], params['pe_b1'][None, :])
    sc = params['pe_bn_g'][None, :] * (1.0 / jnp.sqrt(var_h + EPS))
    sh = params['pe_bn_b'][None, :] - mu_h * sc

    h, tab = _prep_call(x3, pe, A, params['pe_W1'], params['pe_b1'][None, :],
                        sc, sh, params['pe_W2'], params['pe_b2'][None, :],
                        inW[64:], params['in_b'][None, :], etabs[0])

    for li, c in enumerate(params['convs']):
        agg = _sc_call(tab, idx, d0, d1, d2, d3, bounds, zeros_hbm)
        z, s1, s2 = _dense_call(h, agg,
                                c['W1'], c['b1'][None, :],
                                c['W2'], c['b2'][None, :])
        gam = c['bn_g'][None, :]
        bet = c['bn_b'][None, :]
        if li < 3:
            h, tab = _bn_tab_call(z, s1, s2, gam, bet, etabs[li + 1])
        else:
            h = _bn_final_call(z, s1, s2, gam, bet)

    out = _readout_call(h, b3, params['ro_W1'], params['ro_b1'][None, :],
                        roW2, rob2)
    return out[:, 0]
